# Initial kernel scaffold; baseline (speedup 1.0000x reference)
import jax
import jax.numpy as jnp
from jax.experimental import pallas as pl


def kernel(mg_embedding, var_indices, patch_idx):
    raise NotImplementedError("write your pallas kernel here")


# speedup vs baseline: 14.4287x; 14.4287x over previous
import functools

import jax
import jax.numpy as jnp
from jax import lax
from jax.experimental import pallas as pl
from jax.experimental.pallas import tpu as pltpu
from jax.experimental.pallas import tpu_sc as plsc

_NUM_WORKERS = 32
_LANES = 16
_CHUNK = 128


@functools.partial(jax.jit, static_argnames=("interpret",))
def _mg_gather(mg_embedding, var_indices, patch_idx, interpret=False):
    NV, NP, D = mg_embedding.shape
    B, V = var_indices.shape
    P = patch_idx.shape[1]
    R = B * V * P
    r_per_w = R // _NUM_WORKERS
    n_chunks = r_per_w // _CHUNK
    wpb = P // r_per_w

    table = mg_embedding.reshape(NV * NP, D)
    patch_flat = patch_idx.reshape(B * P).astype(jnp.int32)
    var_flat = var_indices.reshape(B * V).astype(jnp.int32)
    var_bcast = jnp.broadcast_to(var_flat[:, None], (B * V, _LANES))

    mesh = plsc.VectorSubcoreMesh(core_axis_name="c", subcore_axis_name="s")

    @functools.partial(
        pl.kernel,
        out_type=jax.ShapeDtypeStruct((R, D), jnp.float32),
        mesh=mesh,
        scratch_types=[
            pltpu.VMEM((_LANES,), jnp.int32),
            pltpu.VMEM((r_per_w,), jnp.int32),
            pltpu.VMEM((n_chunks, _CHUNK), jnp.int32),
            pltpu.VMEM((r_per_w, D), jnp.float32),
            pltpu.SemaphoreType.DMA,
        ],
        interpret=interpret,
    )
    def gather_kernel(table_hbm, varb_hbm, patch_hbm, out_hbm,
                      var_v, pidx_v, idx_v, rows_v, sem):
        wid = lax.axis_index("s") * 2 + lax.axis_index("c")
        bv = wid // wpb
        b = bv // V
        p_off = b * P + (wid % wpb) * r_per_w

        pltpu.sync_copy(varb_hbm.at[bv], var_v)
        pltpu.sync_copy(patch_hbm.at[pl.ds(p_off, r_per_w)], pidx_v)

        voff = var_v[...] * NP

        for i in range(r_per_w // _LANES):
            chunk = pidx_v[pl.ds(i * _LANES, _LANES)] + voff
            idx_v[i * _LANES // _CHUNK,
                  pl.ds((i * _LANES) % _CHUNK, _LANES)] = chunk

        copies = [
            pltpu.async_copy(
                table_hbm.at[idx_v.at[j]],
                rows_v.at[pl.ds(j * _CHUNK, _CHUNK)],
                sem,
            )
            for j in range(n_chunks)
        ]
        for c in copies:
            c.wait()

        pltpu.sync_copy(rows_v, out_hbm.at[pl.ds(wid * r_per_w, r_per_w)])

    out = gather_kernel(table, var_bcast, patch_flat)
    return out.reshape(B, V, 1, P, D)


def kernel(mg_embedding, var_indices, patch_idx):
    return _mg_gather(mg_embedding, var_indices, patch_idx)
